# baseline (speedup 1.0000x reference)
import jax
import jax.numpy as jnp
from jax import lax
from jax.experimental import pallas as pl
from jax.experimental.pallas import tpu as pltpu
from jax.experimental.pallas import tpu_sc as plsc

_NROW = 100000
_D = 64

_NC = 2
_NS = 16
_NW = _NC * _NS
_CH = 400
_CPT = _NROW // _CH
_KPT = 8
_NBUF = 2
_EXTRA = _CPT - (_KPT - 1) * _NW


def _sc_body(i_hbm, oi_hbm, bufs, in_sems, out_sems):
    wid = lax.axis_index("s") * _NC + lax.axis_index("c")
    has_extra = wid < _EXTRA

    def loc(k):
        r = (k * _NW + wid) * _CH
        return jnp.minimum(r, _NROW - _CH)

    def static_valid(k):
        return k != (_KPT - 1)

    def in_cp(k, slot):
        return pltpu.make_async_copy(
            i_hbm.at[pl.ds(loc(k), _CH), :], bufs.at[slot], in_sems.at[slot]
        )

    def out_cp(k, slot):
        return pltpu.make_async_copy(
            bufs.at[slot], oi_hbm.at[pl.ds(loc(k), _CH), :], out_sems.at[slot]
        )

    def guarded(k, fn):
        if static_valid(k):
            fn()
        else:
            @pl.when(has_extra)
            def _():
                fn()

    in_cp(0, 0).start()
    for k in range(_KPT):
        s = k % _NBUF
        nk = k + 1
        if nk < _KPT:
            ns = nk % _NBUF
            if nk >= _NBUF:
                guarded(nk - _NBUF, out_cp(nk - _NBUF, ns).wait)
            guarded(nk, in_cp(nk, ns).start)
        guarded(k, in_cp(k, s).wait)
        guarded(k, out_cp(k, s).start)
    for k in range(_KPT - _NBUF, _KPT):
        guarded(k, out_cp(k, k % _NBUF).wait)


def _sc_copy(embed_item):
    f = pl.kernel(
        _sc_body,
        out_type=jax.ShapeDtypeStruct(embed_item.shape, embed_item.dtype),
        mesh=plsc.VectorSubcoreMesh(core_axis_name="c", subcore_axis_name="s"),
        scratch_types=[
            pltpu.VMEM((_NBUF, _CH, _D), jnp.float32),
            pltpu.SemaphoreType.DMA((_NBUF,)),
            pltpu.SemaphoreType.DMA((_NBUF,)),
        ],
    )
    return f(embed_item)


_TBLK = 5000


def _tc_body(u_ref, ou_ref):
    ou_ref[...] = u_ref[...]


def _tc_copy(embed_user):
    spec = pl.BlockSpec((_TBLK, _D), lambda i: (i, 0))
    return pl.pallas_call(
        _tc_body,
        grid=(_NROW // _TBLK,),
        out_shape=jax.ShapeDtypeStruct(embed_user.shape, embed_user.dtype),
        in_specs=[spec],
        out_specs=spec,
        compiler_params=pltpu.CompilerParams(
            dimension_semantics=("arbitrary",),
        ),
    )(embed_user)


def kernel(embed_user, embed_item):
    out_item = _sc_copy(embed_item)
    out_user = _tc_copy(embed_user)
    return out_item, out_user


# speedup vs baseline: 1.0468x; 1.0468x over previous
import jax
import jax.numpy as jnp
from jax.experimental import pallas as pl
from jax.experimental.pallas import tpu as pltpu

_NROW = 100000
_D = 64
_BLK = 5000


def _copy_body(u_ref, i_ref, oi_ref, ou_ref):
    oi_ref[...] = i_ref[...]
    ou_ref[...] = u_ref[...]


def kernel(embed_user, embed_item):
    spec = pl.BlockSpec((_BLK, _D), lambda i: (i, 0))
    out_shape = (
        jax.ShapeDtypeStruct(embed_item.shape, embed_item.dtype),
        jax.ShapeDtypeStruct(embed_user.shape, embed_user.dtype),
    )
    return pl.pallas_call(
        _copy_body,
        grid=(_NROW // _BLK,),
        out_shape=out_shape,
        in_specs=[spec, spec],
        out_specs=(spec, spec),
        compiler_params=pltpu.CompilerParams(
            dimension_semantics=("arbitrary",),
            needs_layout_passes=True,
        ),
    )(embed_user, embed_item)
